# Initial kernel scaffold; baseline (speedup 1.0000x reference)
import functools

import jax
import jax.numpy as jnp
from jax import lax
from jax.experimental import pallas as pl
from jax.experimental.pallas import tpu as pltpu
from jax.experimental.pallas import tpu_sc as plsc

NUM_EXPERTS = 64
TOP_K = 8
DIM = 1024
HIDDEN_DIM = 512
ROUTE_SCALE = 1.0

NC = 2
NS = 16
NW = NC * NS

BR = 128


def _router_body(x_ref, gwt_ref, bias_ref, sel_ref, w_ref, rank_ref,
                 counts_ref, carry_ref):
    tb = x_ref.shape[0]

    @pl.when(pl.program_id(0) == 0)
    def _():
        carry_ref[...] = jnp.zeros_like(carry_ref)

    xb = x_ref[...]
    scores = jax.nn.sigmoid(
        jnp.dot(xb, gwt_ref[...], preferred_element_type=jnp.float32))
    biased = scores + bias_ref[0:1, :]
    iota_e = lax.broadcasted_iota(jnp.int32, (tb, NUM_EXPERTS), 1)

    cur = biased
    msum = jnp.zeros((tb, NUM_EXPERTS), jnp.int32)
    sel_ks, sc_ks, oh_ks = [], [], []
    for _ in range(TOP_K):
        m = jnp.max(cur, axis=1, keepdims=True)
        idx = jnp.min(jnp.where(cur == m, iota_e, NUM_EXPERTS), axis=1,
                      keepdims=True)
        onehot = iota_e == idx
        sel_ks.append(idx[:, 0])
        sc_ks.append(jnp.sum(jnp.where(onehot, scores, 0.0), axis=1))
        oh_ks.append(onehot)
        msum = msum + onehot.astype(jnp.int32)
        cur = jnp.where(onehot, -jnp.inf, cur)

    sc = jnp.stack(sc_ks, axis=0)
    denom = jnp.maximum(jnp.sum(sc, axis=0, keepdims=True), 1e-20)
    w_ref[...] = sc / denom * ROUTE_SCALE
    sel_ref[...] = jnp.stack(sel_ks, axis=0).astype(jnp.int32)

    carry0 = carry_ref[0:1, :]
    cum = jnp.cumsum(msum, axis=0)
    c_excl = carry0 + cum - msum
    ranks = [jnp.sum(jnp.where(oh_ks[k], c_excl, 0), axis=1)
             for k in range(TOP_K)]
    rank_ref[...] = jnp.stack(ranks, axis=0)
    new_carry = jnp.broadcast_to(carry0 + cum[tb - 1:tb, :], (8, NUM_EXPERTS))
    carry_ref[...] = new_carry
    counts_ref[...] = new_carry


def _router(xf, gwt, bias8):
    t = xf.shape[0]
    tb = 512
    grid = (t // tb,)
    return pl.pallas_call(
        _router_body,
        grid=grid,
        in_specs=[
            pl.BlockSpec((tb, DIM), lambda i: (i, 0)),
            pl.BlockSpec((DIM, NUM_EXPERTS), lambda i: (0, 0)),
            pl.BlockSpec((8, NUM_EXPERTS), lambda i: (0, 0)),
        ],
        out_specs=[
            pl.BlockSpec((TOP_K, tb), lambda i: (0, i)),
            pl.BlockSpec((TOP_K, tb), lambda i: (0, i)),
            pl.BlockSpec((TOP_K, tb), lambda i: (0, i)),
            pl.BlockSpec((8, NUM_EXPERTS), lambda i: (0, 0)),
        ],
        out_shape=[
            jax.ShapeDtypeStruct((TOP_K, t), jnp.int32),
            jax.ShapeDtypeStruct((TOP_K, t), jnp.float32),
            jax.ShapeDtypeStruct((TOP_K, t), jnp.int32),
            jax.ShapeDtypeStruct((8, NUM_EXPERTS), jnp.int32),
        ],
        scratch_shapes=[pltpu.VMEM((8, NUM_EXPERTS), jnp.int32)],
    )(xf, gwt, bias8)


def _dispatch(xf, e_flat, r_flat, w_flat, offset_pad, cap):
    tk = e_flat.shape[0]
    per = tk // NW
    ch = 64
    nch = per // ch
    mesh = plsc.VectorSubcoreMesh(core_axis_name="c", subcore_axis_name="s")

    @functools.partial(
        pl.kernel,
        out_type=[
            jax.ShapeDtypeStruct((cap, DIM), jnp.float32),
            jax.ShapeDtypeStruct((cap, 16), jnp.float32),
            jax.ShapeDtypeStruct((tk,), jnp.int32),
        ],
        mesh=mesh,
        scratch_types=[
            pltpu.VMEM((NUM_EXPERTS,), jnp.int32),
            pltpu.VMEM((ch,), jnp.int32),
            pltpu.VMEM((ch,), jnp.int32),
            pltpu.VMEM((ch,), jnp.float32),
            pltpu.VMEM((ch,), jnp.int32),
            pltpu.VMEM((ch,), jnp.int32),
            pltpu.VMEM((ch, 16), jnp.float32),
            pltpu.VMEM((ch, DIM), jnp.float32),
            pltpu.SemaphoreType.DMA,
        ],
    )
    def dispatch(xf_hbm, e_hbm, r_hbm, w_hbm, off_hbm,
                 perm_hbm, s16_hbm, dest_hbm,
                 off_v, e_v, r_v, w_v, tok_v, dest_v, s16_v, rows_v, sem):
        wid = lax.axis_index("s") * NC + lax.axis_index("c")
        base = wid * per
        pltpu.sync_copy(off_hbm, off_v)
        lane = lax.iota(jnp.int32, 16)

        def chunk_body(ci, carry):
            g = base + ci * ch
            pltpu.sync_copy(e_hbm.at[pl.ds(g, ch)], e_v)
            pltpu.sync_copy(r_hbm.at[pl.ds(g, ch)], r_v)
            pltpu.sync_copy(w_hbm.at[pl.ds(g, ch)], w_v)

            def qbody(q, c2):
                ev = e_v[pl.ds(q * 16, 16)]
                rv = r_v[pl.ds(q * 16, 16)]
                off = plsc.load_gather(off_v, plsc.Indices(ev))
                dest_v[pl.ds(q * 16, 16)] = off + rv
                tok_v[pl.ds(q * 16, 16)] = (g + q * 16 + lane) >> 3
                return c2

            lax.fori_loop(0, ch // 16, qbody, 0)

            def sbody(i, c2):
                idx = jnp.full((16,), i, jnp.int32)
                sp = plsc.load_gather(w_v, plsc.Indices(idx))
                plsc.store_scatter(s16_v, plsc.Indices(idx, lane), sp)
                return c2

            lax.fori_loop(0, ch, sbody, 0)

            pltpu.async_copy(xf_hbm.at[tok_v], rows_v, sem).wait()
            pltpu.async_copy(rows_v, perm_hbm.at[dest_v], sem).wait()
            pltpu.async_copy(s16_v, s16_hbm.at[dest_v], sem).wait()
            pltpu.sync_copy(dest_v, dest_hbm.at[pl.ds(g, ch)])
            return carry

        lax.fori_loop(0, nch, chunk_body, 0)

    return dispatch(xf, e_flat, r_flat, w_flat, offset_pad)


def _expert_body(blk_ref, p_ref, s_ref, w1_ref, w3_ref, w2_ref, o_ref):
    p = p_ref[...] * s_ref[:, 0:1]
    a = jnp.dot(p, w1_ref[0], preferred_element_type=jnp.float32)
    b = jnp.dot(p, w3_ref[0], preferred_element_type=jnp.float32)
    h = a * jax.nn.sigmoid(a) * b
    o_ref[...] = jnp.dot(h, w2_ref[0], preferred_element_type=jnp.float32)


def _experts(blk_expert, perm, s16, w1, w2, w3, nblk):
    grid_spec = pltpu.PrefetchScalarGridSpec(
        num_scalar_prefetch=1,
        grid=(nblk,),
        in_specs=[
            pl.BlockSpec((BR, DIM), lambda i, blk: (i, 0)),
            pl.BlockSpec((BR, 16), lambda i, blk: (i, 0)),
            pl.BlockSpec((1, DIM, HIDDEN_DIM), lambda i, blk: (blk[i], 0, 0)),
            pl.BlockSpec((1, DIM, HIDDEN_DIM), lambda i, blk: (blk[i], 0, 0)),
            pl.BlockSpec((1, HIDDEN_DIM, DIM), lambda i, blk: (blk[i], 0, 0)),
        ],
        out_specs=pl.BlockSpec((BR, DIM), lambda i, blk: (i, 0)),
    )
    return pl.pallas_call(
        _expert_body,
        grid_spec=grid_spec,
        out_shape=jax.ShapeDtypeStruct((nblk * BR, DIM), jnp.float32),
    )(blk_expert, perm, s16, w1, w3, w2)


def _combine(eo, dest, t):
    tok_per = t // NW
    tch = 8
    pch = tch * TOP_K
    nch = tok_per // tch
    mesh = plsc.VectorSubcoreMesh(core_axis_name="c", subcore_axis_name="s")

    @functools.partial(
        pl.kernel,
        out_type=jax.ShapeDtypeStruct((t, DIM), jnp.float32),
        mesh=mesh,
        scratch_types=[
            pltpu.VMEM((pch,), jnp.int32),
            pltpu.VMEM((pch, DIM), jnp.float32),
            pltpu.VMEM((tch, DIM), jnp.float32),
            pltpu.SemaphoreType.DMA,
        ],
    )
    def combine(eo_hbm, dest_hbm, out_hbm, dest_v, rows_v, out_v, sem):
        wid = lax.axis_index("s") * NC + lax.axis_index("c")
        tbase = wid * tok_per

        def chunk_body(ci, carry):
            t0 = tbase + ci * tch
            pltpu.sync_copy(dest_hbm.at[pl.ds(t0 * TOP_K, pch)], dest_v)
            pltpu.async_copy(eo_hbm.at[dest_v], rows_v, sem).wait()

            def cbody(c, c2):
                sl = pl.ds(c * 16, 16)
                for tt in range(tch):
                    acc = rows_v[tt * TOP_K, sl]
                    for j in range(1, TOP_K):
                        acc = acc + rows_v[tt * TOP_K + j, sl]
                    out_v[tt, sl] = acc
                return c2

            lax.fori_loop(0, DIM // 16, cbody, 0)
            pltpu.sync_copy(out_v, out_hbm.at[pl.ds(t0, tch)])
            return carry

        lax.fori_loop(0, nch, chunk_body, 0)

    return combine(eo, dest)


def kernel(x, gate_w, w1, w2, w3, expert_bias):
    bs, slen, dim = x.shape
    xf = x.reshape(-1, dim).astype(jnp.float32)
    t = xf.shape[0]
    tk = t * TOP_K
    nblk = tk // BR + NUM_EXPERTS
    cap = nblk * BR

    gwt = gate_w.T
    bias8 = jnp.broadcast_to(expert_bias[None, :], (8, NUM_EXPERTS))

    sel_t, w_t, rank_t, counts8 = _router(xf, gwt, bias8)
    e_flat = sel_t.T.reshape(-1)
    r_flat = rank_t.T.reshape(-1)
    w_flat = w_t.T.reshape(-1)

    counts = counts8[0]
    nblk_e = (counts + BR - 1) // BR
    offset_pad = (jnp.cumsum(nblk_e) - nblk_e) * BR
    blk_expert = jnp.repeat(
        jnp.arange(NUM_EXPERTS, dtype=jnp.int32), nblk_e,
        total_repeat_length=nblk)

    perm, s16, dest = _dispatch(xf, e_flat, r_flat, w_flat,
                                offset_pad.astype(jnp.int32), cap)
    eo = _experts(blk_expert, perm, s16, w1, w2, w3, nblk)
    out = _combine(eo, dest, t)
    return out.reshape(bs, slen, dim)


# speedup vs baseline: 15.2647x; 15.2647x over previous
import functools

import jax
import jax.numpy as jnp
from jax import lax
from jax.experimental import pallas as pl
from jax.experimental.pallas import tpu as pltpu
from jax.experimental.pallas import tpu_sc as plsc

NUM_EXPERTS = 64
TOP_K = 8
DIM = 1024
HIDDEN_DIM = 512
ROUTE_SCALE = 1.0

NC = 2
NS = 16
NW = NC * NS

BR = 128


def _router_body(x_ref, gwt_ref, bias_ref, sel_ref, w_ref, rank_ref,
                 counts_ref, carry_ref):
    tb = x_ref.shape[0]

    @pl.when(pl.program_id(0) == 0)
    def _():
        carry_ref[...] = jnp.zeros_like(carry_ref)

    xb = x_ref[...]
    scores = jax.nn.sigmoid(
        jnp.dot(xb, gwt_ref[...], preferred_element_type=jnp.float32))
    biased = scores + bias_ref[0:1, :]
    iota_e = lax.broadcasted_iota(jnp.int32, (tb, NUM_EXPERTS), 1)

    cur = biased
    msum = jnp.zeros((tb, NUM_EXPERTS), jnp.float32)
    sel_ks, sc_ks, oh_ks = [], [], []
    for _ in range(TOP_K):
        m = jnp.max(cur, axis=1, keepdims=True)
        idx = jnp.min(jnp.where(cur == m, iota_e, NUM_EXPERTS), axis=1,
                      keepdims=True)
        onehot = iota_e == idx
        sel_ks.append(idx[:, 0])
        sc_ks.append(jnp.sum(jnp.where(onehot, scores, 0.0), axis=1))
        oh_ks.append(onehot)
        msum = msum + onehot.astype(jnp.float32)
        cur = jnp.where(onehot, -jnp.inf, cur)

    sc = jnp.stack(sc_ks, axis=0)
    denom = jnp.maximum(jnp.sum(sc, axis=0, keepdims=True), 1e-20)
    w_ref[...] = sc / denom * ROUTE_SCALE
    sel_ref[...] = jnp.stack(sel_ks, axis=0).astype(jnp.int32)

    carry0 = carry_ref[0:1, :].astype(jnp.float32)
    tri = (lax.broadcasted_iota(jnp.int32, (tb, tb), 0)
           >= lax.broadcasted_iota(jnp.int32, (tb, tb), 1)).astype(jnp.float32)
    cum = jnp.dot(tri, msum, preferred_element_type=jnp.float32)
    c_excl = carry0 + cum - msum
    ranks = [jnp.sum(jnp.where(oh_ks[k], c_excl, 0), axis=1)
             for k in range(TOP_K)]
    rank_ref[...] = jnp.stack(ranks, axis=0).astype(jnp.int32)
    new_carry = jnp.broadcast_to(carry0 + cum[tb - 1:tb, :],
                                 (8, NUM_EXPERTS)).astype(jnp.int32)
    carry_ref[...] = new_carry
    counts_ref[...] = new_carry


def _router(xf, gwt, bias8):
    t = xf.shape[0]
    tb = 512
    grid = (t // tb,)
    return pl.pallas_call(
        _router_body,
        grid=grid,
        in_specs=[
            pl.BlockSpec((tb, DIM), lambda i: (i, 0)),
            pl.BlockSpec((DIM, NUM_EXPERTS), lambda i: (0, 0)),
            pl.BlockSpec((8, NUM_EXPERTS), lambda i: (0, 0)),
        ],
        out_specs=[
            pl.BlockSpec((TOP_K, tb), lambda i: (0, i)),
            pl.BlockSpec((TOP_K, tb), lambda i: (0, i)),
            pl.BlockSpec((TOP_K, tb), lambda i: (0, i)),
            pl.BlockSpec((8, NUM_EXPERTS), lambda i: (0, 0)),
        ],
        out_shape=[
            jax.ShapeDtypeStruct((TOP_K, t), jnp.int32),
            jax.ShapeDtypeStruct((TOP_K, t), jnp.float32),
            jax.ShapeDtypeStruct((TOP_K, t), jnp.int32),
            jax.ShapeDtypeStruct((8, NUM_EXPERTS), jnp.int32),
        ],
        scratch_shapes=[pltpu.VMEM((8, NUM_EXPERTS), jnp.int32)],
    )(xf, gwt, bias8)


def _destcalc_body(e_ref, r_ref, w_ref, off_ref, dest_ref, w16_ref):
    e = e_ref[...]
    acc = r_ref[...]
    for j in range(NUM_EXPERTS):
        acc = acc + jnp.where(e == j, off_ref[j], 0)
    dest_ref[...] = acc
    w16_ref[...] = jnp.broadcast_to(w_ref[...], w16_ref.shape)


def _destcalc(e2, r2, w2, offset_pad):
    rows, cols = e2.shape
    rb = rows // 8
    tkb = rb * cols
    tk = rows * cols
    return pl.pallas_call(
        _destcalc_body,
        grid=(8,),
        in_specs=[
            pl.BlockSpec((rb, cols), lambda i: (i, 0)),
            pl.BlockSpec((rb, cols), lambda i: (i, 0)),
            pl.BlockSpec((tkb, 1), lambda i: (i, 0)),
            pl.BlockSpec(memory_space=pltpu.SMEM),
        ],
        out_specs=[
            pl.BlockSpec((rb, cols), lambda i: (i, 0)),
            pl.BlockSpec((tkb, 128), lambda i: (i, 0)),
        ],
        out_shape=[
            jax.ShapeDtypeStruct((rows, cols), jnp.int32),
            jax.ShapeDtypeStruct((tk, 128), jnp.float32),
        ],
    )(e2, r2, w2, offset_pad)


def _dispatch(xf, dest_flat, w16, cap):
    tk = dest_flat.shape[0]
    per = tk // NW
    ch = 64
    nch = per // ch
    mesh = plsc.VectorSubcoreMesh(core_axis_name="c", subcore_axis_name="s",
                                  num_cores=NC, num_subcores=NS)

    @functools.partial(
        pl.kernel,
        out_type=[
            jax.ShapeDtypeStruct((cap, DIM), jnp.float32),
            jax.ShapeDtypeStruct((cap, 128), jnp.float32),
        ],
        mesh=mesh,
        scratch_types=[
            pltpu.VMEM((ch,), jnp.int32),
            pltpu.VMEM((ch,), jnp.int32),
            pltpu.VMEM((ch, 128), jnp.float32),
            pltpu.VMEM((ch, DIM), jnp.float32),
            pltpu.SemaphoreType.DMA,
        ],
    )
    def dispatch(xf_hbm, dest_hbm, w16_hbm, perm_hbm, s16_hbm,
                 tok_v, dest_v, s16_v, rows_v, sem):
        wid = lax.axis_index("s") * NC + lax.axis_index("c")
        base = wid * per
        lane = lax.iota(jnp.int32, 16)

        def chunk_body(ci, carry):
            g = base + ci * ch
            pltpu.sync_copy(dest_hbm.at[pl.ds(g, ch)], dest_v)
            pltpu.sync_copy(w16_hbm.at[pl.ds(g, ch)], s16_v)

            def qbody(q, c2):
                tok_v[pl.ds(q * 16, 16)] = (g + q * 16 + lane) >> 3
                return c2

            lax.fori_loop(0, ch // 16, qbody, 0)

            pltpu.async_copy(xf_hbm.at[tok_v], rows_v, sem).wait()
            pltpu.async_copy(rows_v, perm_hbm.at[dest_v], sem).wait()
            pltpu.async_copy(s16_v, s16_hbm.at[dest_v], sem).wait()
            return carry

        lax.fori_loop(0, nch, chunk_body, 0)

    return dispatch(xf, dest_flat, w16)


def _expert_body(blk_ref, p_ref, s_ref, w1_ref, w3_ref, w2_ref, o_ref):
    p = p_ref[...] * s_ref[:, 0:1]
    a = jnp.dot(p, w1_ref[0], preferred_element_type=jnp.float32)
    b = jnp.dot(p, w3_ref[0], preferred_element_type=jnp.float32)
    h = a * jax.nn.sigmoid(a) * b
    o_ref[...] = jnp.dot(h, w2_ref[0], preferred_element_type=jnp.float32)


def _experts(blk_expert, perm, s16, w1, w2, w3, nblk):
    grid_spec = pltpu.PrefetchScalarGridSpec(
        num_scalar_prefetch=1,
        grid=(nblk,),
        in_specs=[
            pl.BlockSpec((BR, DIM), lambda i, blk: (i, 0)),
            pl.BlockSpec((BR, 128), lambda i, blk: (i, 0)),
            pl.BlockSpec((1, DIM, HIDDEN_DIM), lambda i, blk: (blk[i], 0, 0)),
            pl.BlockSpec((1, DIM, HIDDEN_DIM), lambda i, blk: (blk[i], 0, 0)),
            pl.BlockSpec((1, HIDDEN_DIM, DIM), lambda i, blk: (blk[i], 0, 0)),
        ],
        out_specs=pl.BlockSpec((BR, DIM), lambda i, blk: (i, 0)),
    )
    return pl.pallas_call(
        _expert_body,
        grid_spec=grid_spec,
        out_shape=jax.ShapeDtypeStruct((nblk * BR, DIM), jnp.float32),
    )(blk_expert, perm, s16, w1, w3, w2)


def _combine(eo, dest, t):
    tok_per = t // NW
    tch = 8
    pch = tch * TOP_K
    nch = tok_per // tch
    mesh = plsc.VectorSubcoreMesh(core_axis_name="c", subcore_axis_name="s", num_cores=NC, num_subcores=NS)

    @functools.partial(
        pl.kernel,
        out_type=jax.ShapeDtypeStruct((t, DIM), jnp.float32),
        mesh=mesh,
        scratch_types=[
            pltpu.VMEM((pch,), jnp.int32),
            pltpu.VMEM((pch, DIM), jnp.float32),
            pltpu.VMEM((tch, DIM), jnp.float32),
            pltpu.SemaphoreType.DMA,
        ],
    )
    def combine(eo_hbm, dest_hbm, out_hbm, dest_v, rows_v, out_v, sem):
        wid = lax.axis_index("s") * NC + lax.axis_index("c")
        tbase = wid * tok_per

        def chunk_body(ci, carry):
            t0 = tbase + ci * tch
            pltpu.sync_copy(dest_hbm.at[pl.ds(t0 * TOP_K, pch)], dest_v)
            pltpu.async_copy(eo_hbm.at[dest_v], rows_v, sem).wait()

            def cbody(c, c2):
                sl = pl.ds(c * 16, 16)
                for tt in range(tch):
                    acc = rows_v[tt * TOP_K, sl]
                    for j in range(1, TOP_K):
                        acc = acc + rows_v[tt * TOP_K + j, sl]
                    out_v[tt, sl] = acc
                return c2

            lax.fori_loop(0, DIM // 16, cbody, 0)
            pltpu.sync_copy(out_v, out_hbm.at[pl.ds(t0, tch)])
            return carry

        lax.fori_loop(0, nch, chunk_body, 0)

    return combine(eo, dest)


def kernel(x, gate_w, w1, w2, w3, expert_bias):
    bs, slen, dim = x.shape
    xf = x.reshape(-1, dim).astype(jnp.float32)
    t = xf.shape[0]
    tk = t * TOP_K
    nblk = tk // BR + NUM_EXPERTS
    cap = nblk * BR

    gwt = gate_w.T
    bias8 = jnp.broadcast_to(expert_bias[None, :], (8, NUM_EXPERTS))

    sel_t, w_t, rank_t, counts8 = _router(xf, gwt, bias8)
    e_flat = sel_t.T.reshape(-1)
    r_flat = rank_t.T.reshape(-1)
    w_flat = w_t.T.reshape(-1)

    counts = counts8[0]
    nblk_e = (counts + BR - 1) // BR
    offset_pad = ((jnp.cumsum(nblk_e) - nblk_e) * BR).astype(jnp.int32)
    blk_expert = jnp.repeat(
        jnp.arange(NUM_EXPERTS, dtype=jnp.int32), nblk_e,
        total_repeat_length=nblk)

    dest2, w16 = _destcalc(e_flat.reshape(-1, 128), r_flat.reshape(-1, 128),
                           w_flat.reshape(-1, 1), offset_pad)
    dest_flat = dest2.reshape(-1)

    perm, s16 = _dispatch(xf, dest_flat, w16, cap)
    eo = _experts(blk_expert, perm, s16, w1, w2, w3, nblk)
    out = _combine(eo, dest_flat, t)
    return out.reshape(bs, slen, dim)


# speedup vs baseline: 17.1935x; 1.1264x over previous
import functools

import jax
import jax.numpy as jnp
from jax import lax
from jax.experimental import pallas as pl
from jax.experimental.pallas import tpu as pltpu
from jax.experimental.pallas import tpu_sc as plsc

NUM_EXPERTS = 64
TOP_K = 8
DIM = 1024
HIDDEN_DIM = 512
ROUTE_SCALE = 1.0

NC = 2
NS = 16
NW = NC * NS

BR = 128


def _router_body(x_ref, gwt_ref, bias_ref, sel_ref, w_ref, rank_ref,
                 counts_ref, carry_ref):
    tb = x_ref.shape[0]

    @pl.when(pl.program_id(0) == 0)
    def _():
        carry_ref[...] = jnp.zeros_like(carry_ref)

    xb = x_ref[...]
    scores = jax.nn.sigmoid(
        jnp.dot(xb, gwt_ref[...], preferred_element_type=jnp.float32))
    biased = scores + bias_ref[0:1, :]
    iota_e = lax.broadcasted_iota(jnp.int32, (tb, NUM_EXPERTS), 1)

    cur = biased
    msum = jnp.zeros((tb, NUM_EXPERTS), jnp.float32)
    sel_ks, sc_ks, oh_ks = [], [], []
    for _ in range(TOP_K):
        m = jnp.max(cur, axis=1, keepdims=True)
        idx = jnp.min(jnp.where(cur == m, iota_e, NUM_EXPERTS), axis=1,
                      keepdims=True)
        onehot = iota_e == idx
        sel_ks.append(idx[:, 0])
        sc_ks.append(jnp.sum(jnp.where(onehot, scores, 0.0), axis=1))
        oh_ks.append(onehot)
        msum = msum + onehot.astype(jnp.float32)
        cur = jnp.where(onehot, -jnp.inf, cur)

    sc = jnp.stack(sc_ks, axis=0)
    denom = jnp.maximum(jnp.sum(sc, axis=0, keepdims=True), 1e-20)
    w_ref[...] = sc / denom * ROUTE_SCALE
    sel_ref[...] = jnp.stack(sel_ks, axis=0).astype(jnp.int32)

    carry0 = carry_ref[0:1, :].astype(jnp.float32)
    tri = (lax.broadcasted_iota(jnp.int32, (tb, tb), 0)
           >= lax.broadcasted_iota(jnp.int32, (tb, tb), 1)).astype(jnp.float32)
    cum = jnp.dot(tri, msum, preferred_element_type=jnp.float32)
    c_excl = carry0 + cum - msum
    ranks = [jnp.sum(jnp.where(oh_ks[k], c_excl, 0), axis=1)
             for k in range(TOP_K)]
    rank_ref[...] = jnp.stack(ranks, axis=0).astype(jnp.int32)
    new_carry = jnp.broadcast_to(carry0 + cum[tb - 1:tb, :],
                                 (8, NUM_EXPERTS)).astype(jnp.int32)
    carry_ref[...] = new_carry
    counts_ref[...] = new_carry


def _router(xf, gwt, bias8):
    t = xf.shape[0]
    tb = 512
    grid = (t // tb,)
    return pl.pallas_call(
        _router_body,
        grid=grid,
        in_specs=[
            pl.BlockSpec((tb, DIM), lambda i: (i, 0)),
            pl.BlockSpec((DIM, NUM_EXPERTS), lambda i: (0, 0)),
            pl.BlockSpec((8, NUM_EXPERTS), lambda i: (0, 0)),
        ],
        out_specs=[
            pl.BlockSpec((TOP_K, tb), lambda i: (0, i)),
            pl.BlockSpec((TOP_K, tb), lambda i: (0, i)),
            pl.BlockSpec((TOP_K, tb), lambda i: (0, i)),
            pl.BlockSpec((8, NUM_EXPERTS), lambda i: (0, 0)),
        ],
        out_shape=[
            jax.ShapeDtypeStruct((TOP_K, t), jnp.int32),
            jax.ShapeDtypeStruct((TOP_K, t), jnp.float32),
            jax.ShapeDtypeStruct((TOP_K, t), jnp.int32),
            jax.ShapeDtypeStruct((8, NUM_EXPERTS), jnp.int32),
        ],
        scratch_shapes=[pltpu.VMEM((8, NUM_EXPERTS), jnp.int32)],
    )(xf, gwt, bias8)


def _destcalc_body(e_ref, r_ref, w_ref, off_ref, dest_ref, w16_ref):
    e = e_ref[...]
    acc = r_ref[...]
    for j in range(NUM_EXPERTS):
        acc = acc + jnp.where(e == j, off_ref[j], 0)
    dest_ref[...] = acc
    w16_ref[...] = jnp.broadcast_to(w_ref[...], w16_ref.shape)


def _destcalc(e_t, r_t, w2, offset_pad):
    k, t = e_t.shape
    tb = t // 8
    tk = k * t
    tkb = tk // 8
    return pl.pallas_call(
        _destcalc_body,
        grid=(8,),
        in_specs=[
            pl.BlockSpec((k, tb), lambda i: (0, i)),
            pl.BlockSpec((k, tb), lambda i: (0, i)),
            pl.BlockSpec((tkb, 1), lambda i: (i, 0)),
            pl.BlockSpec(memory_space=pltpu.SMEM),
        ],
        out_specs=[
            pl.BlockSpec((k, tb), lambda i: (0, i)),
            pl.BlockSpec((tkb, 128), lambda i: (i, 0)),
        ],
        out_shape=[
            jax.ShapeDtypeStruct((k, t), jnp.int32),
            jax.ShapeDtypeStruct((tk, 128), jnp.float32),
        ],
    )(e_t, r_t, w2, offset_pad)


def _dispatch(xf, dest_t, w16, cap):
    t = xf.shape[0]
    tok_per = t // NW
    tch = 64
    nch = tok_per // tch
    mesh = plsc.VectorSubcoreMesh(core_axis_name="c", subcore_axis_name="s",
                                  num_cores=NC, num_subcores=NS)

    @functools.partial(
        pl.kernel,
        out_type=[
            jax.ShapeDtypeStruct((cap, DIM), jnp.float32),
            jax.ShapeDtypeStruct((cap, 128), jnp.float32),
        ],
        mesh=mesh,
        scratch_types=[
            pltpu.VMEM((TOP_K, tch), jnp.int32),
            pltpu.VMEM((tch, 128), jnp.float32),
            pltpu.VMEM((tch, 128), jnp.float32),
            pltpu.VMEM((tch, DIM), jnp.float32),
            pltpu.SemaphoreType.DMA,
            pltpu.SemaphoreType.DMA,
        ],
    )
    def dispatch(xf_hbm, dt_hbm, w16_hbm, perm_hbm, s16_hbm,
                 idx_v, s16a_v, s16b_v, rows_v, sem, sem2):
        wid = lax.axis_index("s") * NC + lax.axis_index("c")
        tbase = wid * tok_per

        def chunk_body(ci, carry):
            t0 = tbase + ci * tch
            pltpu.sync_copy(xf_hbm.at[pl.ds(t0, tch)], rows_v)
            for k in range(TOP_K):
                pltpu.sync_copy(dt_hbm.at[k, pl.ds(t0, tch)], idx_v.at[k])
            cps = []
            for k in range(TOP_K):
                cps.append(
                    pltpu.async_copy(rows_v, perm_hbm.at[idx_v.at[k]], sem))
            bufs = (s16a_v, s16b_v)
            cps2 = []
            for k in range(TOP_K):
                sb = bufs[k % 2]
                if k >= 2:
                    cps2[k - 2].wait()
                pltpu.sync_copy(w16_hbm.at[pl.ds(k * t + t0, tch)], sb)
                cps2.append(
                    pltpu.async_copy(sb, s16_hbm.at[idx_v.at[k]], sem2))
            cps2[TOP_K - 2].wait()
            cps2[TOP_K - 1].wait()
            for cp in cps:
                cp.wait()
            return carry

        lax.fori_loop(0, nch, chunk_body, 0)

    return dispatch(xf, dest_t, w16)


def _expert_body(blk_ref, p_ref, s_ref, w1_ref, w3_ref, w2_ref, o_ref):
    p = p_ref[...] * s_ref[:, 0:1]
    a = jnp.dot(p, w1_ref[0], preferred_element_type=jnp.float32)
    b = jnp.dot(p, w3_ref[0], preferred_element_type=jnp.float32)
    h = a * jax.nn.sigmoid(a) * b
    o_ref[...] = jnp.dot(h, w2_ref[0], preferred_element_type=jnp.float32)


def _experts(blk_expert, perm, s16, w1, w2, w3, nblk):
    grid_spec = pltpu.PrefetchScalarGridSpec(
        num_scalar_prefetch=1,
        grid=(nblk,),
        in_specs=[
            pl.BlockSpec((BR, DIM), lambda i, blk: (i, 0)),
            pl.BlockSpec((BR, 128), lambda i, blk: (i, 0)),
            pl.BlockSpec((1, DIM, HIDDEN_DIM), lambda i, blk: (blk[i], 0, 0)),
            pl.BlockSpec((1, DIM, HIDDEN_DIM), lambda i, blk: (blk[i], 0, 0)),
            pl.BlockSpec((1, HIDDEN_DIM, DIM), lambda i, blk: (blk[i], 0, 0)),
        ],
        out_specs=pl.BlockSpec((BR, DIM), lambda i, blk: (i, 0)),
    )
    return pl.pallas_call(
        _expert_body,
        grid_spec=grid_spec,
        out_shape=jax.ShapeDtypeStruct((nblk * BR, DIM), jnp.float32),
    )(blk_expert, perm, s16, w1, w3, w2)


def _combine(eo, dest_t, t):
    tok_per = t // NW
    tch = 8
    nch = tok_per // tch
    mesh = plsc.VectorSubcoreMesh(core_axis_name="c", subcore_axis_name="s",
                                  num_cores=NC, num_subcores=NS)

    @functools.partial(
        pl.kernel,
        out_type=jax.ShapeDtypeStruct((t, DIM), jnp.float32),
        mesh=mesh,
        scratch_types=[
            pltpu.VMEM((TOP_K, tok_per), jnp.int32),
            pltpu.VMEM((TOP_K * tch, DIM), jnp.float32),
            pltpu.VMEM((tch, DIM), jnp.float32),
            pltpu.SemaphoreType.DMA,
        ],
    )
    def combine(eo_hbm, dt_hbm, out_hbm, idx_v, rows_v, out_v, sem):
        wid = lax.axis_index("s") * NC + lax.axis_index("c")
        tbase = wid * tok_per
        for k in range(TOP_K):
            pltpu.sync_copy(dt_hbm.at[k, pl.ds(tbase, tok_per)], idx_v.at[k])

        def chunk_body(ci, carry):
            c0 = ci * tch
            cps = []
            for k in range(TOP_K):
                cps.append(pltpu.async_copy(
                    eo_hbm.at[idx_v.at[k, pl.ds(c0, tch)]],
                    rows_v.at[pl.ds(k * tch, tch)], sem))
            for cp in cps:
                cp.wait()

            def cbody(c, c2):
                sl = pl.ds(c * 16, 16)
                for tt in range(tch):
                    acc = rows_v[tt, sl]
                    for j in range(1, TOP_K):
                        acc = acc + rows_v[j * tch + tt, sl]
                    out_v[tt, sl] = acc
                return c2

            lax.fori_loop(0, DIM // 16, cbody, 0)
            pltpu.sync_copy(out_v, out_hbm.at[pl.ds(tbase + c0, tch)])
            return carry

        lax.fori_loop(0, nch, chunk_body, 0)

    return combine(eo, dest_t)


def kernel(x, gate_w, w1, w2, w3, expert_bias):
    bs, slen, dim = x.shape
    xf = x.reshape(-1, dim).astype(jnp.float32)
    t = xf.shape[0]
    tk = t * TOP_K
    nblk = tk // BR + NUM_EXPERTS
    cap = nblk * BR

    gwt = gate_w.T
    bias8 = jnp.broadcast_to(expert_bias[None, :], (8, NUM_EXPERTS))

    sel_t, w_t, rank_t, counts8 = _router(xf, gwt, bias8)

    counts = counts8[0]
    nblk_e = (counts + BR - 1) // BR
    offset_pad = ((jnp.cumsum(nblk_e) - nblk_e) * BR).astype(jnp.int32)
    blk_expert = jnp.repeat(
        jnp.arange(NUM_EXPERTS, dtype=jnp.int32), nblk_e,
        total_repeat_length=nblk)

    dest_t, w16 = _destcalc(sel_t, rank_t, w_t.reshape(-1, 1), offset_pad)

    perm, s16 = _dispatch(xf, dest_t, w16, cap)
    eo = _experts(blk_expert, perm, s16, w1, w2, w3, nblk)
    out = _combine(eo, dest_t, t)
    return out.reshape(bs, slen, dim)
